# Initial kernel scaffold; baseline (speedup 1.0000x reference)
import jax
import jax.numpy as jnp
from jax.experimental import pallas as pl


def kernel(features, probablity_weak, memory_bank, segmentation, ignore_mask):
    raise NotImplementedError("write your pallas kernel here")


# speedup vs baseline: 4.9502x; 4.9502x over previous
import functools

import jax
import jax.numpy as jnp
from jax import lax
from jax.experimental import pallas as pl
from jax.experimental.pallas import tpu as pltpu
from jax.experimental.pallas import tpu_sc as plsc

B, C, H, W = 8, 256, 128, 128
HW = H * W
NPIX = B * HW
NCLS = 21
NBINS = NCLS + 1
NW = 32
P = NPIX // NW
KC = 2
NCHUNK = C // KC
BANK = NBINS * C
CNTB = 128
L = 16
MOM = 0.99


def _seg_sums_sc(feats2d, seg, prob, ig):
  mesh = plsc.VectorSubcoreMesh(core_axis_name="c", subcore_axis_name="s")

  @functools.partial(
      pl.kernel,
      mesh=mesh,
      compiler_params=pltpu.CompilerParams(needs_layout_passes=False),
      out_type=[
          jax.ShapeDtypeStruct((NW, BANK), jnp.float32),
          jax.ShapeDtypeStruct((NW, CNTB), jnp.float32),
      ],
      scratch_types=[
          pltpu.VMEM((P,), jnp.int32),
          pltpu.VMEM((P,), jnp.float32),
          pltpu.VMEM((P,), jnp.int32),
          pltpu.VMEM((P + L,), jnp.int32),
          pltpu.VMEM((P + L,), jnp.int32),
          pltpu.VMEM((L * BANK,), jnp.float32),
          pltpu.VMEM((L * CNTB,), jnp.float32),
          pltpu.VMEM((KC, P), jnp.float32),
          pltpu.VMEM((KC, P), jnp.float32),
          pltpu.SemaphoreType.DMA,
          pltpu.SemaphoreType.DMA,
      ],
  )
  def k(feats_hbm, seg_hbm, prob_hbm, ig_hbm, out_sums, out_cnt,
        seg_v, prob_v, ig_v, pix_v, bad_v, acc_v, cnt_v, fb0, fb1,
        sem0, sem1):
    wid = lax.axis_index("s") * 2 + lax.axis_index("c")
    b = wid // 4
    q = wid % 4
    pixbase = wid * P
    lane = lax.iota(jnp.int32, L)
    zf = jnp.zeros((L,), jnp.float32)

    pltpu.sync_copy(seg_hbm.at[pl.ds(pixbase, P)], seg_v)
    pltpu.sync_copy(prob_hbm.at[pl.ds(pixbase, P)], prob_v)
    pltpu.sync_copy(ig_hbm.at[pl.ds(pixbase, P)], ig_v)

    def zero_acc(i, carry):
      acc_v[pl.ds(i * L, L)] = zf
      return carry

    lax.fori_loop(0, L * BANK // L, zero_acc, jnp.int32(0))

    def zero_cnt(i, carry):
      cnt_v[pl.ds(i * L, L)] = zf
      return carry

    lax.fori_loop(0, L * CNTB // L, zero_cnt, jnp.int32(0))

    ones = jnp.ones((L,), jnp.float32)

    def compact(i, cn):
      s = seg_v[pl.ds(i * L, L)]
      pr = prob_v[pl.ds(i * L, L)]
      im = ig_v[pl.ds(i * L, L)]
      valid = (pr > 0.95) & (im != 255) & (s >= 0) & (s < NCLS)
      binv = jnp.where(valid, s, NCLS)
      plsc.addupdate_scatter(cnt_v, [lane * CNTB + binv], ones)
      plsc.store_compressed(pix_v.at[pl.ds(cn, L)], lane + i * L, mask=valid)
      plsc.store_compressed(bad_v.at[pl.ds(cn, L)], binv * C, mask=valid)
      return cn + jnp.sum(valid.astype(jnp.int32))

    count = lax.fori_loop(0, P // L, compact, jnp.int32(0))
    pix_v[pl.ds(count, L)] = jnp.zeros((L,), jnp.int32)
    bad_v[pl.ds(count, L)] = jnp.full((L,), NCLS * C, jnp.int32)
    nvec = (count + L - 1) // L

    lane_bank = lane * BANK

    def feat_copy(c, buf, sem):
      r0 = b * C + c * KC
      return pltpu.make_async_copy(
          feats_hbm.at[pl.ds(r0, KC), pl.ds(q * P, P)], buf, sem)

    def gather_chunk(c, buf):
      def g_body(j, carry):
        pv = pix_v[pl.ds(j * L, L)]
        bv = bad_v[pl.ds(j * L, L)]
        addr = lane_bank + bv
        for cl in range(KC):
          val = plsc.load_gather(buf, [jnp.full((L,), cl, jnp.int32), pv])
          plsc.addupdate_scatter(acc_v, [addr + (c * KC + cl)], val)
        return carry

      lax.fori_loop(0, nvec, g_body, jnp.int32(0))

    feat_copy(0, fb0, sem0).start()

    def ch_body(j2, carry):
      c0 = 2 * j2
      c1 = 2 * j2 + 1
      feat_copy(c1, fb1, sem1).start()
      feat_copy(c0, fb0, sem0).wait()
      gather_chunk(c0, fb0)

      @pl.when(j2 < NCHUNK // 2 - 1)
      def _():
        feat_copy(c0 + 2, fb0, sem0).start()

      feat_copy(c1, fb1, sem1).wait()
      gather_chunk(c1, fb1)
      return carry

    lax.fori_loop(0, NCHUNK // 2, ch_body, jnp.int32(0))

    def red_sums(j, carry):
      v = acc_v[pl.ds(j * L, L)]
      for l in range(1, L):
        v = v + acc_v[pl.ds(l * BANK + j * L, L)]
      acc_v[pl.ds(j * L, L)] = v
      return carry

    lax.fori_loop(0, BANK // L, red_sums, jnp.int32(0))

    def red_cnt(j, carry):
      v = cnt_v[pl.ds(j * L, L)]
      for l in range(1, L):
        v = v + cnt_v[pl.ds(l * CNTB + j * L, L)]
      cnt_v[pl.ds(j * L, L)] = v
      return carry

    lax.fori_loop(0, CNTB // L, red_cnt, jnp.int32(0))

    pltpu.sync_copy(acc_v.at[pl.ds(0, BANK)], out_sums.at[wid])
    pltpu.sync_copy(cnt_v.at[pl.ds(0, CNTB)], out_cnt.at[wid])

  return k(feats2d, seg, prob, ig)


def _combine_body(s_ref, c_ref, b_ref, o_ref):
  s = jnp.sum(s_ref[...], axis=0)
  cn = jnp.sum(c_ref[...], axis=1, keepdims=True)
  s21 = s[:NCLS]
  c21 = cn[:NCLS]
  mean = s21 / jnp.maximum(c21, 1.0)
  present = c21 > 0.0
  row = b_ref[...]
  nz = jnp.sum((row == 0.0).astype(jnp.float32), axis=1, keepdims=True)
  is_zero = nz == float(C)
  do_copy = present & is_zero
  idx = lax.broadcasted_iota(jnp.int32, (NCLS, 1), 0)
  first = jnp.min(jnp.where(do_copy, idx, jnp.int32(2**30)))
  need = idx <= first
  do_mom = present & (~is_zero) & need
  mom_row = MOM * row + (1.0 - MOM) * mean
  o_ref[...] = jnp.where(do_copy, mean, jnp.where(do_mom, mom_row, row))


def _combine_tc(sums3, cnt_t, bank2):
  return pl.pallas_call(
      _combine_body,
      out_shape=jax.ShapeDtypeStruct((NCLS, C), jnp.float32),
  )(sums3, cnt_t, bank2)


def kernel(features, probablity_weak, memory_bank, segmentation, ignore_mask):
  feats2d = features.reshape(B * C, HW)
  seg = segmentation.reshape(NPIX)
  prob = probablity_weak.reshape(NPIX)
  ig = ignore_mask.reshape(NPIX)
  sums, cnts = _seg_sums_sc(feats2d, seg, prob, ig)
  out = _combine_tc(sums.reshape(NW, NBINS, C), cnts.T, memory_bank.reshape(NCLS, C))
  return out.reshape(NCLS, 1, C)


# speedup vs baseline: 10.0683x; 2.0339x over previous
import functools

import jax
import jax.numpy as jnp
from jax import lax
from jax.experimental import pallas as pl
from jax.experimental.pallas import tpu as pltpu
from jax.experimental.pallas import tpu_sc as plsc

B, C, H, W = 8, 256, 128, 128
HW = H * W
NPIX = B * HW
NCLS = 21
NBINS = NCLS + 1
NW = 32
CT = C // 4
BANKT = NBINS * CT
CNTB = 128
L = 16
SR = 4096
NROUND = HW // SR
NBUF = 4
MOM = 0.99


def _seg_sums_sc(feats2d, seg, prob, ig):
  mesh = plsc.VectorSubcoreMesh(core_axis_name="c", subcore_axis_name="s")

  @functools.partial(
      pl.kernel,
      mesh=mesh,
      compiler_params=pltpu.CompilerParams(needs_layout_passes=False),
      out_type=[
          jax.ShapeDtypeStruct((NW, BANKT), jnp.float32),
          jax.ShapeDtypeStruct((NW, CNTB), jnp.float32),
      ],
      scratch_types=[
          pltpu.VMEM((SR,), jnp.int32),
          pltpu.VMEM((SR,), jnp.float32),
          pltpu.VMEM((SR,), jnp.int32),
          pltpu.VMEM((HW + 2 * L,), jnp.int32),
          pltpu.VMEM((L * BANKT,), jnp.float32),
          pltpu.VMEM((L * CNTB,), jnp.float32),
          pltpu.VMEM((NBUF * HW,), jnp.float32),
          pltpu.SemaphoreType.DMA,
          pltpu.SemaphoreType.DMA,
          pltpu.SemaphoreType.DMA,
          pltpu.SemaphoreType.DMA,
          pltpu.SemaphoreType.DMA,
      ],
  )
  def k(feats_hbm, seg_hbm, prob_hbm, ig_hbm, out_sums, out_cnt,
        seg_v, prob_v, ig_v, pk_v, acc_v, cnt_v, fb,
        sem0, sem1, sem2, sem3, sem_in):
    wid = lax.axis_index("s") * 2 + lax.axis_index("c")
    b = wid // 4
    cq = wid % 4
    lane = lax.iota(jnp.int32, L)
    zf = jnp.zeros((L,), jnp.float32)
    sems = [sem0, sem1, sem2, sem3]

    def feat_copy(c, u, sem):
      return pltpu.make_async_copy(
          feats_hbm.at[pl.ds((b * C + cq * CT + c) * HW, HW)],
          fb.at[pl.ds(u * HW, HW)], sem)

    for u in range(NBUF - 1):
      feat_copy(u, u, sems[u]).start()

    def zero_acc(i, carry):
      acc_v[pl.ds(i * L, L)] = zf
      return carry

    lax.fori_loop(0, L * BANKT // L, zero_acc, jnp.int32(0))

    def zero_cnt(i, carry):
      cnt_v[pl.ds(i * L, L)] = zf
      return carry

    lax.fori_loop(0, L * CNTB // L, zero_cnt, jnp.int32(0))

    ones = jnp.ones((L,), jnp.float32)
    count = jnp.int32(0)
    for r in range(NROUND):
      base = b * HW + r * SR
      pltpu.make_async_copy(seg_hbm.at[pl.ds(base, SR)], seg_v, sem_in).start()
      pltpu.make_async_copy(prob_hbm.at[pl.ds(base, SR)], prob_v, sem_in).start()
      pltpu.make_async_copy(ig_hbm.at[pl.ds(base, SR)], ig_v, sem_in).start()
      pltpu.make_async_copy(seg_hbm.at[pl.ds(base, SR)], seg_v, sem_in).wait()
      pltpu.make_async_copy(prob_hbm.at[pl.ds(base, SR)], prob_v, sem_in).wait()
      pltpu.make_async_copy(ig_hbm.at[pl.ds(base, SR)], ig_v, sem_in).wait()

      def compact(i, cn, r=r):
        s = seg_v[pl.ds(i * L, L)]
        pr = prob_v[pl.ds(i * L, L)]
        im = ig_v[pl.ds(i * L, L)]
        valid = (pr > 0.95) & (im != 255) & (s >= 0) & (s < NCLS)
        binv = jnp.where(valid, s, NCLS)
        plsc.addupdate_scatter(cnt_v, [lane * CNTB + binv], ones)
        packed = binv * HW + (lane + (r * SR + i * L))
        plsc.store_compressed(pk_v.at[pl.ds(cn, L)], packed, mask=valid)
        return cn + jnp.sum(valid.astype(jnp.int32))

      count = lax.fori_loop(0, SR // L, compact, count)

    dead = jnp.full((L,), NCLS * HW, jnp.int32)
    pk_v[pl.ds(count, L)] = dead
    pk_v[pl.ds(count + L, L)] = dead
    nvec2 = (count + 2 * L - 1) // (2 * L)

    lane_bank = lane * BANKT

    def gather_chunk(c, u):
      base_v = lane_bank + c
      buf = fb.at[pl.ds(u * HW, HW)]

      def g_body(j, carry):
        for h in range(2):
          w = pk_v[pl.ds(j * 2 * L + h * L, L)]
          pix = w & jnp.int32(HW - 1)
          binoff = lax.shift_right_logical(w & jnp.int32(~(HW - 1)), 8)
          val = plsc.load_gather(buf, [pix])
          plsc.addupdate_scatter(acc_v, [base_v + binoff], val)
        return carry

      lax.fori_loop(0, nvec2, g_body, jnp.int32(0))

    def ch_body(j4, carry):
      for u in range(NBUF):
        c = NBUF * j4 + u
        nxt = c + NBUF
        feat_copy(c, u, sems[u]).wait()
        gather_chunk(c, u)

        @pl.when(nxt < CT)
        def _():
          feat_copy(nxt, u, sems[u]).start()

      return carry

    feat_copy(NBUF - 1, NBUF - 1, sems[NBUF - 1]).start()
    lax.fori_loop(0, CT // NBUF, ch_body, jnp.int32(0))

    def red_sums(j, carry):
      v = acc_v[pl.ds(j * L, L)]
      for l in range(1, L):
        v = v + acc_v[pl.ds(l * BANKT + j * L, L)]
      acc_v[pl.ds(j * L, L)] = v
      return carry

    lax.fori_loop(0, BANKT // L, red_sums, jnp.int32(0))

    def red_cnt(j, carry):
      v = cnt_v[pl.ds(j * L, L)]
      for l in range(1, L):
        v = v + cnt_v[pl.ds(l * CNTB + j * L, L)]
      cnt_v[pl.ds(j * L, L)] = v
      return carry

    lax.fori_loop(0, CNTB // L, red_cnt, jnp.int32(0))

    @pl.when(cq != 0)
    def _():
      def rez(j, carry):
        cnt_v[pl.ds(j * L, L)] = zf
        return carry

      lax.fori_loop(0, CNTB // L, rez, jnp.int32(0))

    pltpu.sync_copy(acc_v.at[pl.ds(0, BANKT)], out_sums.at[wid])
    pltpu.sync_copy(cnt_v.at[pl.ds(0, CNTB)], out_cnt.at[wid])

  return k(feats2d, seg, prob, ig)


def _combine_body(s_ref, c_ref, b_ref, o_ref):
  s = jnp.sum(s_ref[...], axis=0)
  cn = jnp.sum(c_ref[...], axis=1, keepdims=True)
  s21 = s[:NCLS]
  c21 = cn[:NCLS]
  mean = s21 / jnp.maximum(c21, 1.0)
  present = c21 > 0.0
  row = b_ref[...]
  nz = jnp.sum((row == 0.0).astype(jnp.float32), axis=1, keepdims=True)
  is_zero = nz == float(C)
  do_copy = present & is_zero
  idx = lax.broadcasted_iota(jnp.int32, (NCLS, 1), 0)
  first = jnp.min(jnp.where(do_copy, idx, jnp.int32(2**30)))
  need = idx <= first
  do_mom = present & (~is_zero) & need
  mom_row = MOM * row + (1.0 - MOM) * mean
  o_ref[...] = jnp.where(do_copy, mean, jnp.where(do_mom, mom_row, row))


def _combine_tc(sums3, cnt_t, bank2):
  return pl.pallas_call(
      _combine_body,
      out_shape=jax.ShapeDtypeStruct((NCLS, C), jnp.float32),
  )(sums3, cnt_t, bank2)


def kernel(features, probablity_weak, memory_bank, segmentation, ignore_mask):
  feats2d = features.reshape(B * C * HW)
  seg = segmentation.reshape(NPIX)
  prob = probablity_weak.reshape(NPIX)
  ig = ignore_mask.reshape(NPIX)
  sums, cnts = _seg_sums_sc(feats2d, seg, prob, ig)
  sums_b = sums.reshape(B, 4, NBINS, CT).transpose(0, 2, 1, 3).reshape(B, NBINS, C)
  out = _combine_tc(sums_b, cnts.T, memory_bank.reshape(NCLS, C))
  return out.reshape(NCLS, 1, C)
